# baseline (speedup 1.0000x reference)
import functools
import jax
import jax.numpy as jnp
from jax import lax
from jax.experimental import pallas as pl
from jax.experimental.pallas import tpu as pltpu
from jax.experimental.pallas import tpu_sc as plsc

N = 10000
D = 128
KHOPS = 3
NC, NS = 2, 16
NW = NC * NS
CHUNK = 128
NA = 10112
SLAB = NA // NS
ROWBLK = 1000


def _mesh():
    return plsc.VectorSubcoreMesh(core_axis_name="c", subcore_axis_name="s")


NBUF = 1
NSUB = CHUNK // 16


def _unpack_idx(pvec, j, dst, shift, mask):
    for t in range(NSUB):
        v = pvec[j, pl.ds(t * 16, 16)]
        dst[0, pl.ds(t * 16, 16)] = lax.shift_right_logical(v, shift) & mask


def _make_sc_degree(n_chunks):
    @functools.partial(
        pl.kernel,
        out_type=jax.ShapeDtypeStruct((NC, NA, D), jnp.float32),
        mesh=_mesh(),
        scratch_types=[
            pltpu.VMEM((n_chunks, CHUNK), jnp.int32),
            pltpu.VMEM((1, CHUNK), jnp.int32),
            pltpu.VMEM((CHUNK, D), jnp.float32),
            pltpu.VMEM_SHARED((NA, D), jnp.float32),
        ],
    )
    def deg_kernel(pidx_hbm, ones_hbm, zeros_hbm, out_hbm, pvec, cbuf, ones_v, acc):
        cid = lax.axis_index("c")
        sid = lax.axis_index("s")
        wid = cid * NS + sid
        pltpu.sync_copy(pidx_hbm.at[wid], pvec)
        pltpu.sync_copy(ones_hbm, ones_v)
        r0 = sid * SLAB
        pltpu.sync_copy(zeros_hbm.at[pl.ds(r0, SLAB)], acc.at[pl.ds(r0, SLAB)])
        plsc.subcore_barrier()

        def body(j, carry):
            _unpack_idx(pvec, j, cbuf, 0, 16383)
            pltpu.sync_copy(ones_v, acc.at[cbuf.at[0]], add=True)
            return carry

        lax.fori_loop(0, n_chunks, body, 0)
        plsc.subcore_barrier()
        pltpu.sync_copy(acc.at[pl.ds(r0, SLAB)], out_hbm.at[cid, pl.ds(r0, SLAB)])

    return deg_kernel


SUPER = 2


def _make_sc_hop(n_chunks):
    assert n_chunks % SUPER == 0

    @functools.partial(
        pl.kernel,
        out_type=jax.ShapeDtypeStruct((NC, NA, D), jnp.float32),
        mesh=_mesh(),
        scratch_types=[
            pltpu.VMEM((n_chunks, CHUNK), jnp.int32),
            pltpu.VMEM((SUPER * CHUNK,), jnp.int32),
            pltpu.VMEM((SUPER * CHUNK,), jnp.int32),
            pltpu.VMEM((SUPER * CHUNK, D), jnp.float32),
            pltpu.VMEM_SHARED((NA, D), jnp.float32),
            pltpu.SemaphoreType.DMA,
        ],
    )
    def hop_kernel(g_hbm, pidx_hbm, zeros_hbm, out_hbm,
                   pvec, rbuf, cbuf, rows, acc, sem):
        cid = lax.axis_index("c")
        sid = lax.axis_index("s")
        wid = cid * NS + sid
        pltpu.sync_copy(pidx_hbm.at[wid], pvec)
        r0 = sid * SLAB
        pltpu.sync_copy(zeros_hbm.at[pl.ds(r0, SLAB)], acc.at[pl.ds(r0, SLAB)])
        plsc.subcore_barrier()

        def body(i, carry):
            for q in range(SUPER):
                for t in range(NSUB):
                    v = pvec[i * SUPER + q, pl.ds(t * 16, 16)]
                    rbuf[pl.ds(q * CHUNK + t * 16, 16)] = (
                        lax.shift_right_logical(v, 14) & 16383)
                    cbuf[pl.ds(q * CHUNK + t * 16, 16)] = v & 16383
            pltpu.async_copy(g_hbm.at[rbuf], rows, sem).wait()
            pltpu.sync_copy(rows, acc.at[cbuf], add=True)
            return carry

        lax.fori_loop(0, n_chunks // SUPER, body, 0)
        plsc.subcore_barrier()
        pltpu.sync_copy(acc.at[pl.ds(r0, SLAB)], out_hbm.at[cid, pl.ds(r0, SLAB)])

    return hop_kernel


def _rowspec():
    return pl.BlockSpec((ROWBLK, D), lambda i: (i, 0))


def _wspec():
    return pl.BlockSpec((D, D), lambda i: (0, 0))


def _tc_prep_body(x_ref, d0_ref, d1_ref, g_ref, dinv_ref):
    deg = d0_ref[:, :1] + d1_ref[:, :1]
    dinv = jnp.where(deg > 0, lax.rsqrt(deg), 0.0)
    dinv_b = jnp.broadcast_to(dinv, (ROWBLK, D))
    dinv_ref[...] = dinv_b
    g_ref[...] = x_ref[...] * dinv_b


def _tc_prep(x, d0, d1):
    return pl.pallas_call(
        _tc_prep_body,
        grid=(N // ROWBLK,),
        in_specs=[_rowspec(), _rowspec(), _rowspec()],
        out_specs=[_rowspec(), _rowspec()],
        out_shape=[jax.ShapeDtypeStruct((N, D), jnp.float32)] * 2,
    )(x, d0, d1)


def _tc_w0_body(x_ref, w_ref, acc_ref):
    acc_ref[...] = jnp.dot(x_ref[...], w_ref[...],
                           preferred_element_type=jnp.float32)


def _tc_w0(x, w0):
    return pl.pallas_call(
        _tc_w0_body,
        grid=(N // ROWBLK,),
        in_specs=[_rowspec(), _wspec()],
        out_specs=_rowspec(),
        out_shape=jax.ShapeDtypeStruct((N, D), jnp.float32),
    )(x, w0)


def _tc_g_body(s0_ref, s1_ref, dinv_ref, g_ref):
    dinv = dinv_ref[...]
    g_ref[...] = (s0_ref[...] + s1_ref[...]) * dinv * dinv


def _tc_g(s0, s1, dinv):
    return pl.pallas_call(
        _tc_g_body,
        grid=(N // ROWBLK,),
        in_specs=[_rowspec(), _rowspec(), _rowspec()],
        out_specs=_rowspec(),
        out_shape=jax.ShapeDtypeStruct((N, D), jnp.float32),
    )(s0, s1, dinv)


def _tc_acc_body(s0_ref, s1_ref, dinv_ref, w_ref, accin_ref, acc_ref):
    h = (s0_ref[...] + s1_ref[...]) * dinv_ref[...]
    acc_ref[...] = accin_ref[...] + jnp.dot(
        h, w_ref[...], preferred_element_type=jnp.float32)


def _tc_acc(s0, s1, dinv, wk, acc):
    return pl.pallas_call(
        _tc_acc_body,
        grid=(N // ROWBLK,),
        in_specs=[_rowspec(), _rowspec(), _rowspec(), _wspec(), _rowspec()],
        out_specs=_rowspec(),
        out_shape=jax.ShapeDtypeStruct((N, D), jnp.float32),
    )(s0, s1, dinv, wk, acc)


def _tc_final_body(s0_ref, s1_ref, dinv_ref, w_ref, accin_ref, b_ref, o_ref):
    h = (s0_ref[...] + s1_ref[...]) * dinv_ref[...]
    o = accin_ref[...] + jnp.dot(h, w_ref[...], preferred_element_type=jnp.float32)
    o_ref[...] = jnp.maximum(o + b_ref[...], 0.0)


def _tc_final(s0, s1, dinv, wk, acc, bias):
    return pl.pallas_call(
        _tc_final_body,
        grid=(N // ROWBLK,),
        in_specs=[_rowspec(), _rowspec(), _rowspec(), _wspec(), _rowspec(),
                  pl.BlockSpec((1, D), lambda i: (0, 0))],
        out_specs=_rowspec(),
        out_shape=jax.ShapeDtypeStruct((N, D), jnp.float32),
    )(s0, s1, dinv, wk, acc, bias)


def kernel(x, edge_index, edge_attr, batch, Ws, bias):
    del edge_attr, batch
    e = edge_index.shape[1]
    row = edge_index[0].astype(jnp.int32)
    col = edge_index[1].astype(jnp.int32)

    per_tile = -(-e // (NW * CHUNK * SUPER)) * CHUNK * SUPER
    n_chunks = per_tile // CHUNK
    if e % NW == 0:
        ept = e // NW
        dcol = N + (jnp.arange(per_tile - ept, dtype=jnp.int32) % (NA - N))
        rowp = jnp.concatenate(
            [row.reshape(NW, ept),
             jnp.zeros((NW, per_tile - ept), jnp.int32)], axis=1)
        colp = jnp.concatenate(
            [col.reshape(NW, ept),
             jnp.broadcast_to(dcol, (NW, per_tile - ept))], axis=1)
    else:
        epad = per_tile * NW - e
        rowp = jnp.concatenate([row, jnp.zeros((epad,), jnp.int32)])
        colp = jnp.concatenate(
            [col, N + (jnp.arange(epad, dtype=jnp.int32) % (NA - N))])
    pidx = ((rowp.reshape(-1) << 14) | colp.reshape(-1)).reshape(
        NW, n_chunks, CHUNK)

    zeros_d = jnp.zeros((NA, D), jnp.float32)
    ones_d = jnp.ones((CHUNK, D), jnp.float32)

    deg_parts = _make_sc_degree(n_chunks)(pidx, ones_d, zeros_d)
    acc = _tc_w0(x, Ws[0])
    d0 = deg_parts[0, :N, :]
    d1 = deg_parts[1, :N, :]
    g, dinv = _tc_prep(x, d0, d1)

    hop = _make_sc_hop(n_chunks)
    for k in range(1, KHOPS + 1):
        s = hop(g, pidx, zeros_d)
        s0 = s[0, :N, :]
        s1 = s[1, :N, :]
        if k < KHOPS:
            g = _tc_g(s0, s1, dinv)
            acc = _tc_acc(s0, s1, dinv, Ws[k], acc)
        else:
            out = _tc_final(s0, s1, dinv, Ws[k], acc, bias.reshape(1, D))
    return out


# speedup vs baseline: 1.3522x; 1.3522x over previous
import functools
import jax
import jax.numpy as jnp
from jax import lax
from jax.experimental import pallas as pl
from jax.experimental.pallas import tpu as pltpu
from jax.experimental.pallas import tpu_sc as plsc

N = 10000
D = 128
KHOPS = 3
NC, NS = 2, 16
NW = NC * NS
CHUNK = 128
NA = 10112
SLAB = NA // NS
ROWBLK = 1000


def _mesh():
    return plsc.VectorSubcoreMesh(core_axis_name="c", subcore_axis_name="s")


NBUF = 1
NSUB = CHUNK // 16


def _unpack_idx(pvec, j, dst, shift, mask):
    for t in range(NSUB):
        v = pvec[j, pl.ds(t * 16, 16)]
        dst[0, pl.ds(t * 16, 16)] = lax.shift_right_logical(v, shift) & mask


def _make_sc_degree(n_chunks):
    @functools.partial(
        pl.kernel,
        out_type=jax.ShapeDtypeStruct((NC, NA, D), jnp.float32),
        mesh=_mesh(),
        scratch_types=[
            pltpu.VMEM((n_chunks, CHUNK), jnp.int32),
            pltpu.VMEM((1, CHUNK), jnp.int32),
            pltpu.VMEM((CHUNK, D), jnp.float32),
            pltpu.VMEM_SHARED((NA, D), jnp.float32),
        ],
    )
    def deg_kernel(pidx_hbm, ones_hbm, zeros_hbm, out_hbm, pvec, cbuf, ones_v, acc):
        cid = lax.axis_index("c")
        sid = lax.axis_index("s")
        wid = cid * NS + sid
        pltpu.sync_copy(pidx_hbm.at[wid], pvec)
        pltpu.sync_copy(ones_hbm, ones_v)
        r0 = sid * SLAB
        pltpu.sync_copy(zeros_hbm.at[pl.ds(r0, SLAB)], acc.at[pl.ds(r0, SLAB)])
        plsc.subcore_barrier()

        def body(j, carry):
            _unpack_idx(pvec, j, cbuf, 0, 16383)
            pltpu.sync_copy(ones_v, acc.at[cbuf.at[0]], add=True)
            return carry

        lax.fori_loop(0, n_chunks, body, 0)
        plsc.subcore_barrier()
        pltpu.sync_copy(acc.at[pl.ds(r0, SLAB)], out_hbm.at[cid, pl.ds(r0, SLAB)])

    return deg_kernel


def _make_sc_hop(n_chunks):
    @functools.partial(
        pl.kernel,
        out_type=jax.ShapeDtypeStruct((NC, NA, D), jnp.float32),
        mesh=_mesh(),
        scratch_types=[
            pltpu.VMEM((n_chunks, CHUNK), jnp.int32),
            pltpu.VMEM((1, CHUNK), jnp.int32),
            pltpu.VMEM((1, CHUNK), jnp.int32),
            pltpu.VMEM((CHUNK, D), jnp.float32),
            pltpu.VMEM_SHARED((NA, D), jnp.float32),
            pltpu.SemaphoreType.DMA,
        ],
    )
    def hop_kernel(g_hbm, pidx_hbm, zeros_hbm, out_hbm,
                   pvec, rbuf, cbuf, rows, acc, sem):
        cid = lax.axis_index("c")
        sid = lax.axis_index("s")
        wid = cid * NS + sid
        pltpu.sync_copy(pidx_hbm.at[wid], pvec)
        r0 = sid * SLAB
        pltpu.sync_copy(zeros_hbm.at[pl.ds(r0, SLAB)], acc.at[pl.ds(r0, SLAB)])
        plsc.subcore_barrier()

        def body(j, carry):
            _unpack_idx(pvec, j, rbuf, 14, 16383)
            pltpu.async_copy(g_hbm.at[rbuf.at[0]], rows, sem).wait()
            _unpack_idx(pvec, j, cbuf, 0, 16383)
            pltpu.sync_copy(rows, acc.at[cbuf.at[0]], add=True)
            return carry

        lax.fori_loop(0, n_chunks, body, 0)
        plsc.subcore_barrier()
        pltpu.sync_copy(acc.at[pl.ds(r0, SLAB)], out_hbm.at[cid, pl.ds(r0, SLAB)])

    return hop_kernel


def _rowspec():
    return pl.BlockSpec((ROWBLK, D), lambda i: (i, 0))


def _wspec():
    return pl.BlockSpec((D, D), lambda i: (0, 0))


def _tc_prep_body(x_ref, d0_ref, d1_ref, g_ref, dinv_ref):
    deg = d0_ref[:, :1] + d1_ref[:, :1]
    dinv = jnp.where(deg > 0, lax.rsqrt(deg), 0.0)
    dinv_b = jnp.broadcast_to(dinv, (ROWBLK, D))
    dinv_ref[...] = dinv_b
    g_ref[...] = x_ref[...] * dinv_b


def _tc_prep(x, d0, d1):
    return pl.pallas_call(
        _tc_prep_body,
        grid=(N // ROWBLK,),
        in_specs=[_rowspec(), _rowspec(), _rowspec()],
        out_specs=[_rowspec(), _rowspec()],
        out_shape=[jax.ShapeDtypeStruct((N, D), jnp.float32)] * 2,
    )(x, d0, d1)


def _tc_w0_body(x_ref, w_ref, acc_ref):
    acc_ref[...] = jnp.dot(x_ref[...], w_ref[...],
                           preferred_element_type=jnp.float32)


def _tc_w0(x, w0):
    return pl.pallas_call(
        _tc_w0_body,
        grid=(N // ROWBLK,),
        in_specs=[_rowspec(), _wspec()],
        out_specs=_rowspec(),
        out_shape=jax.ShapeDtypeStruct((N, D), jnp.float32),
    )(x, w0)


def _tc_g_body(s0_ref, s1_ref, dinv_ref, g_ref):
    dinv = dinv_ref[...]
    g_ref[...] = (s0_ref[...] + s1_ref[...]) * dinv * dinv


def _tc_g(s0, s1, dinv):
    return pl.pallas_call(
        _tc_g_body,
        grid=(N // ROWBLK,),
        in_specs=[_rowspec(), _rowspec(), _rowspec()],
        out_specs=_rowspec(),
        out_shape=jax.ShapeDtypeStruct((N, D), jnp.float32),
    )(s0, s1, dinv)


def _tc_acc_body(s0_ref, s1_ref, dinv_ref, w_ref, accin_ref, acc_ref):
    h = (s0_ref[...] + s1_ref[...]) * dinv_ref[...]
    acc_ref[...] = accin_ref[...] + jnp.dot(
        h, w_ref[...], preferred_element_type=jnp.float32)


def _tc_acc(s0, s1, dinv, wk, acc):
    return pl.pallas_call(
        _tc_acc_body,
        grid=(N // ROWBLK,),
        in_specs=[_rowspec(), _rowspec(), _rowspec(), _wspec(), _rowspec()],
        out_specs=_rowspec(),
        out_shape=jax.ShapeDtypeStruct((N, D), jnp.float32),
    )(s0, s1, dinv, wk, acc)


def _tc_final_body(s0_ref, s1_ref, dinv_ref, w_ref, accin_ref, b_ref, o_ref):
    h = (s0_ref[...] + s1_ref[...]) * dinv_ref[...]
    o = accin_ref[...] + jnp.dot(h, w_ref[...], preferred_element_type=jnp.float32)
    o_ref[...] = jnp.maximum(o + b_ref[...], 0.0)


def _tc_final(s0, s1, dinv, wk, acc, bias):
    return pl.pallas_call(
        _tc_final_body,
        grid=(N // ROWBLK,),
        in_specs=[_rowspec(), _rowspec(), _rowspec(), _wspec(), _rowspec(),
                  pl.BlockSpec((1, D), lambda i: (0, 0))],
        out_specs=_rowspec(),
        out_shape=jax.ShapeDtypeStruct((N, D), jnp.float32),
    )(s0, s1, dinv, wk, acc, bias)


def kernel(x, edge_index, edge_attr, batch, Ws, bias):
    del edge_attr, batch
    e = edge_index.shape[1]
    row = edge_index[0].astype(jnp.int32)
    col = edge_index[1].astype(jnp.int32)

    per_tile = -(-e // (NW * CHUNK)) * CHUNK
    n_chunks = per_tile // CHUNK
    if e % NW == 0:
        ept = e // NW
        dcol = N + (jnp.arange(per_tile - ept, dtype=jnp.int32) % (NA - N))
        rowp = jnp.concatenate(
            [row.reshape(NW, ept),
             jnp.zeros((NW, per_tile - ept), jnp.int32)], axis=1)
        colp = jnp.concatenate(
            [col.reshape(NW, ept),
             jnp.broadcast_to(dcol, (NW, per_tile - ept))], axis=1)
    else:
        epad = per_tile * NW - e
        rowp = jnp.concatenate([row, jnp.zeros((epad,), jnp.int32)])
        colp = jnp.concatenate(
            [col, N + (jnp.arange(epad, dtype=jnp.int32) % (NA - N))])
    pidx = ((rowp.reshape(-1) << 14) | colp.reshape(-1)).reshape(
        NW, n_chunks, CHUNK)

    zeros_d = jnp.zeros((NA, D), jnp.float32)
    ones_d = jnp.ones((CHUNK, D), jnp.float32)

    deg_parts = _make_sc_degree(n_chunks)(pidx, ones_d, zeros_d)
    acc = _tc_w0(x, Ws[0])
    d0 = deg_parts[0, :N, :]
    d1 = deg_parts[1, :N, :]
    g, dinv = _tc_prep(x, d0, d1)

    hop = _make_sc_hop(n_chunks)
    for k in range(1, KHOPS + 1):
        s = hop(g, pidx, zeros_d)
        s0 = s[0, :N, :]
        s1 = s[1, :N, :]
        if k < KHOPS:
            g = _tc_g(s0, s1, dinv)
            acc = _tc_acc(s0, s1, dinv, Ws[k], acc)
        else:
            out = _tc_final(s0, s1, dinv, Ws[k], acc, bias.reshape(1, D))
    return out
